# baseline (speedup 1.0000x reference)
import functools

import jax
import jax.numpy as jnp
from jax import lax
from jax.experimental import pallas as pl
from jax.experimental.pallas import tpu as pltpu
from jax.experimental.pallas import tpu_sc as plsc

N_NODES = 10000
N_EDGES = 320000
D = 128

NC = 2
NS = 16
NW = NC * NS
CHUNK = 128
NCH = 80
E_PAD = NW * NCH * CHUNK
ACC_ROWS = N_NODES + 8
ROWS_PER_TILE = 624
TAIL_ROWS = N_NODES - NS * ROWS_PER_TILE
TAIL_OFF = NS * ROWS_PER_TILE


def _lin1_body(x_ref, w_ref, b_ref, o_ref):
    o_ref[...] = (
        jnp.dot(x_ref[...], w_ref[...], preferred_element_type=jnp.float32)
        + b_ref[...]
    )


_lin1 = pl.pallas_call(
    _lin1_body,
    grid=(10,),
    in_specs=[
        pl.BlockSpec((1000, D), lambda i: (i, 0)),
        pl.BlockSpec((D, D), lambda i: (0, 0)),
        pl.BlockSpec((1, D), lambda i: (0, 0)),
    ],
    out_specs=pl.BlockSpec((1000, D), lambda i: (i, 0)),
    out_shape=jax.ShapeDtypeStruct((N_NODES, D), jnp.float32),
)


def _sc_body(h_hbm, src_hbm, dst_hbm, z_hbm, out_hbm,
             dst_v, src_a, src_b, rows_a, rows_b, acc,
             isem_a, isem_b, gsem_a, gsem_b):
    c = lax.axis_index("c")
    s = lax.axis_index("s")
    wid = c * NS + s

    pltpu.sync_copy(dst_hbm.at[wid], dst_v)

    pltpu.sync_copy(z_hbm, acc.at[pl.ds(s * ROWS_PER_TILE, ROWS_PER_TILE)])
    @pl.when(s == 0)
    def _():
        pltpu.sync_copy(z_hbm.at[pl.ds(0, TAIL_ROWS)],
                        acc.at[pl.ds(TAIL_OFF, TAIL_ROWS)])
    plsc.subcore_barrier()

    pltpu.async_copy(src_hbm.at[wid, 0], src_a, isem_a)

    def step(j, cur, oth):
        src_c, rows_c, isem_c, gsem_c = cur
        src_o, rows_o, isem_o, gsem_o = oth
        pltpu.make_async_copy(src_hbm.at[wid, j], src_c, isem_c).wait()
        pltpu.async_copy(h_hbm.at[src_c.at[0]], rows_c, gsem_c)

        @pl.when(j > 0)
        def _():
            pltpu.make_async_copy(h_hbm.at[src_o.at[0]], rows_o, gsem_o).wait()

        @pl.when(j + 1 < NCH)
        def _():
            pltpu.async_copy(src_hbm.at[wid, j + 1], src_o, isem_o)

        @pl.when(j > 0)
        def _():
            pltpu.sync_copy(rows_o, acc.at[dst_v.at[j - 1]], add=True)

    buf_a = (src_a, rows_a, isem_a, gsem_a)
    buf_b = (src_b, rows_b, isem_b, gsem_b)

    def body(jj, carry):
        step(2 * jj, buf_a, buf_b)
        step(2 * jj + 1, buf_b, buf_a)
        return carry

    lax.fori_loop(0, NCH // 2, body, 0)
    pltpu.make_async_copy(h_hbm.at[src_b.at[0]], rows_b, gsem_b).wait()
    pltpu.sync_copy(rows_b, acc.at[dst_v.at[NCH - 1]], add=True)
    plsc.subcore_barrier()

    pltpu.sync_copy(
        acc.at[pl.ds(s * ROWS_PER_TILE, ROWS_PER_TILE)],
        out_hbm.at[c].at[pl.ds(s * ROWS_PER_TILE, ROWS_PER_TILE)],
    )
    @pl.when(s == 0)
    def _():
        pltpu.sync_copy(acc.at[pl.ds(TAIL_OFF, TAIL_ROWS)],
                        out_hbm.at[c].at[pl.ds(TAIL_OFF, TAIL_ROWS)])


_sc_scatter = functools.partial(
    pl.kernel,
    out_type=jax.ShapeDtypeStruct((NC, N_NODES, D), jnp.float32),
    mesh=plsc.VectorSubcoreMesh(core_axis_name="c", subcore_axis_name="s"),
    scratch_types=[
        pltpu.VMEM((NCH, CHUNK), jnp.int32),
        pltpu.VMEM((1, CHUNK), jnp.int32),
        pltpu.VMEM((1, CHUNK), jnp.int32),
        pltpu.VMEM((CHUNK, D), jnp.float32),
        pltpu.VMEM((CHUNK, D), jnp.float32),
        pltpu.VMEM_SHARED((ACC_ROWS, D), jnp.float32),
        pltpu.SemaphoreType.DMA,
        pltpu.SemaphoreType.DMA,
        pltpu.SemaphoreType.DMA,
        pltpu.SemaphoreType.DMA,
    ],
)(_sc_body)


def _lin2_body(p_ref, w_ref, b_ref, o_ref):
    a = jnp.maximum(p_ref[0] + p_ref[1], 0.0)
    o_ref[...] = (
        jnp.dot(a, w_ref[...], preferred_element_type=jnp.float32) + b_ref[...]
    )


_lin2 = pl.pallas_call(
    _lin2_body,
    grid=(10,),
    in_specs=[
        pl.BlockSpec((NC, 1000, D), lambda i: (0, i, 0)),
        pl.BlockSpec((D, D), lambda i: (0, 0)),
        pl.BlockSpec((1, D), lambda i: (0, 0)),
    ],
    out_specs=pl.BlockSpec((1000, D), lambda i: (i, 0)),
    out_shape=jax.ShapeDtypeStruct((N_NODES, D), jnp.float32),
)


def kernel(x, edge_index, W1, b1, W2, b2):
    src = edge_index[0].astype(jnp.int32)
    dst = edge_index[1].astype(jnp.int32)
    npad = E_PAD - N_EDGES
    src = jnp.concatenate([src, jnp.zeros((npad,), jnp.int32)])
    dst = jnp.concatenate([dst, jnp.full((npad,), N_NODES, jnp.int32)])
    src = src.reshape(NW, NCH, 1, CHUNK)
    dst = dst.reshape(NW, NCH, CHUNK)
    zeros = jnp.zeros((ROWS_PER_TILE, D), jnp.float32)
    h = _lin1(x, W1.T, b1.reshape(1, D))
    partials = _sc_scatter(h, src, dst, zeros)
    return _lin2(partials, W2.T, b2.reshape(1, D))


# speedup vs baseline: 3.0184x; 3.0184x over previous
import functools

import jax
import jax.numpy as jnp
from jax import lax
from jax.experimental import pallas as pl
from jax.experimental.pallas import tpu as pltpu
from jax.experimental.pallas import tpu_sc as plsc

N_NODES = 10000
N_EDGES = 320000
D = 128

NC = 2
NS = 16
NW = NC * NS
CHUNK = 100
NCH = 100
NQ = 4
QCH = NCH // NQ
ROWS_PER_TILE = 624
TAIL_ROWS = N_NODES - NS * ROWS_PER_TILE
TAIL_OFF = NS * ROWS_PER_TILE


def _lin1_body(x_ref, w_ref, b_ref, o_ref):
    o_ref[...] = (
        jnp.dot(x_ref[...], w_ref[...], preferred_element_type=jnp.float32)
        + b_ref[...]
    )


_lin1 = pl.pallas_call(
    _lin1_body,
    grid=(10,),
    in_specs=[
        pl.BlockSpec((1000, D), lambda i: (i, 0)),
        pl.BlockSpec((D, D), lambda i: (0, 0)),
        pl.BlockSpec((1, D), lambda i: (0, 0)),
    ],
    out_specs=pl.BlockSpec((1000, D), lambda i: (i, 0)),
    out_shape=jax.ShapeDtypeStruct((N_NODES, D), jnp.float32),
)


def _sc_body(h_hbm, src_hbm, dst_hbm, z_hbm, out_hbm,
             dst_v, srcq_a, srcq_b, rows_a, rows_b, acc,
             qsem_a, qsem_b, gsem_a, gsem_b):
    c = lax.axis_index("c")
    s = lax.axis_index("s")
    wid = c * NS + s

    pltpu.sync_copy(dst_hbm.at[wid], dst_v)

    pltpu.sync_copy(z_hbm, acc.at[pl.ds(s * ROWS_PER_TILE, ROWS_PER_TILE)])
    @pl.when(s == 0)
    def _():
        pltpu.sync_copy(z_hbm.at[pl.ds(0, TAIL_ROWS)],
                        acc.at[pl.ds(TAIL_OFF, TAIL_ROWS)])
    plsc.subcore_barrier()

    qbufs = (srcq_a, srcq_b)
    qsems = (qsem_a, qsem_b)

    pltpu.async_copy(src_hbm.at[wid, 0], srcq_a, qsem_a)

    for q in range(NQ):
        qbuf, qsem = qbufs[q % 2], qsems[q % 2]
        pltpu.make_async_copy(src_hbm.at[wid, q], qbuf, qsem).wait()
        if q + 1 < NQ:
            pltpu.async_copy(src_hbm.at[wid, q + 1],
                             qbufs[(q + 1) % 2], qsems[(q + 1) % 2])

        base = q * QCH
        pltpu.async_copy(h_hbm.at[qbuf.at[0]], rows_a, gsem_a)

        def pair(m, carry, qbuf=qbuf):
            k = 2 * m
            pltpu.async_copy(h_hbm.at[qbuf.at[k + 1]], rows_b, gsem_b)
            pltpu.make_async_copy(h_hbm.at[qbuf.at[k]], rows_a, gsem_a).wait()
            pltpu.sync_copy(rows_a, acc.at[dst_v.at[base + k]], add=True)

            @pl.when(k + 2 < QCH)
            def _():
                pltpu.async_copy(h_hbm.at[qbuf.at[k + 2]], rows_a, gsem_a)
            pltpu.make_async_copy(h_hbm.at[qbuf.at[k + 1]], rows_b,
                                  gsem_b).wait()
            pltpu.sync_copy(rows_b, acc.at[dst_v.at[base + k + 1]], add=True)
            return carry

        lax.fori_loop(0, QCH // 2, pair, 0)
        pltpu.make_async_copy(h_hbm.at[qbuf.at[QCH - 1]], rows_a,
                              gsem_a).wait()
        pltpu.sync_copy(rows_a, acc.at[dst_v.at[base + QCH - 1]], add=True)

    plsc.subcore_barrier()

    pltpu.sync_copy(
        acc.at[pl.ds(s * ROWS_PER_TILE, ROWS_PER_TILE)],
        out_hbm.at[c].at[pl.ds(s * ROWS_PER_TILE, ROWS_PER_TILE)],
    )
    @pl.when(s == 0)
    def _():
        pltpu.sync_copy(acc.at[pl.ds(TAIL_OFF, TAIL_ROWS)],
                        out_hbm.at[c].at[pl.ds(TAIL_OFF, TAIL_ROWS)])


_sc_scatter = functools.partial(
    pl.kernel,
    out_type=jax.ShapeDtypeStruct((NC, N_NODES, D), jnp.float32),
    mesh=plsc.VectorSubcoreMesh(core_axis_name="c", subcore_axis_name="s"),
    scratch_types=[
        pltpu.VMEM((NCH, CHUNK), jnp.int32),
        pltpu.VMEM((QCH, CHUNK), jnp.int32),
        pltpu.VMEM((QCH, CHUNK), jnp.int32),
        pltpu.VMEM((CHUNK, D), jnp.float32),
        pltpu.VMEM((CHUNK, D), jnp.float32),
        pltpu.VMEM_SHARED((N_NODES, D), jnp.float32),
        pltpu.SemaphoreType.DMA,
        pltpu.SemaphoreType.DMA,
        pltpu.SemaphoreType.DMA,
        pltpu.SemaphoreType.DMA,
    ],
)(_sc_body)


def _lin2_body(p_ref, w_ref, b_ref, o_ref):
    a = jnp.maximum(p_ref[0] + p_ref[1], 0.0)
    o_ref[...] = (
        jnp.dot(a, w_ref[...], preferred_element_type=jnp.float32) + b_ref[...]
    )


_lin2 = pl.pallas_call(
    _lin2_body,
    grid=(10,),
    in_specs=[
        pl.BlockSpec((NC, 1000, D), lambda i: (0, i, 0)),
        pl.BlockSpec((D, D), lambda i: (0, 0)),
        pl.BlockSpec((1, D), lambda i: (0, 0)),
    ],
    out_specs=pl.BlockSpec((1000, D), lambda i: (i, 0)),
    out_shape=jax.ShapeDtypeStruct((N_NODES, D), jnp.float32),
)


def kernel(x, edge_index, W1, b1, W2, b2):
    src = edge_index[0].astype(jnp.int32).reshape(NW, NQ, QCH, CHUNK)
    dst = edge_index[1].astype(jnp.int32).reshape(NW, NCH, CHUNK)
    zeros = jnp.zeros((ROWS_PER_TILE, D), jnp.float32)
    h = _lin1(x, W1.T, b1.reshape(1, D))
    partials = _sc_scatter(h, src, dst, zeros)
    return _lin2(partials, W2.T, b2.reshape(1, D))
